# baseline (speedup 1.0000x reference)
import functools

import jax
import jax.numpy as jnp
import numpy as np
from jax import lax
from jax.experimental import pallas as pl
from jax.experimental.pallas import tpu as pltpu
from jax.experimental.pallas import tpu_sc as plsc

NC = 2
NS = 16
NW = NC * NS
L = 16

H = 224
W = 224
OUT = H // 2
B = 4
C = 96
TC_C = 32
SC_C = C - TC_C
SC_IMG = B * SC_C
RB = 28
NBLK_W = SC_IMG * 4 // NW
GROUPS = OUT // L
HB = 8


def _fuzzy_math(x0, x1, x2, x3):
    m_inner = (x1 + x2) * 0.5
    m_all = ((x0 + x1) + (x2 + x3)) * 0.25
    v_avg = (m_inner + m_all) * 0.5
    w0 = jnp.abs(x0 - v_avg)
    w1 = jnp.abs(x1 - v_avg)
    w2 = jnp.abs(x2 - v_avg)
    w3 = jnp.abs(x3 - v_avg)
    s0 = (w1 + w2) * 0.5 + 1e-4
    s1 = ((w0 + w1) + (w2 + w3)) * 0.25 + 1e-4

    r0 = 1.0 / s0
    r1 = 1.0 / s1
    c0 = (r0 * r0) * -0.5
    c1 = (r1 * r1) * -0.5

    def gauss(xm, mu, c):
        d = xm - mu
        return jnp.exp((d * d) * c)

    p00 = gauss(x0, m_inner, c0)
    p0i = gauss(x1, m_inner, c0)
    p03 = gauss(x3, m_inner, c0)
    p10 = gauss(x0, m_all, c1)
    p11 = gauss(x1, m_all, c1)
    p12 = gauss(x2, m_all, c1)
    p13 = gauss(x3, m_all, c1)

    thresh = jnp.minimum(
        jnp.minimum(jnp.maximum(p00, p10), jnp.maximum(p0i, p11)),
        jnp.minimum(jnp.maximum(p0i, p12), jnp.maximum(p03, p13)))
    q0 = p00 + p10
    q1 = p0i + p11
    q2 = p0i + p12
    q3 = p03 + p13

    primary = q1 >= (thresh + thresh)
    secondary = jnp.logical_and(jnp.logical_not(primary), s1 < 0.001)
    num = (q0 * x0 + q1 * x1) + (q2 * x2 + q3 * x3)
    den = (q0 + q1) + (q2 + q3)
    denoised = num / den
    return jnp.where(primary, m_all, jnp.where(secondary, v_avg, denoised))


def _fuzzy_body(x_hbm, o_hbm, in0, in1, out0, out1, si0, si1, so0, so1):
    wid = lax.axis_index("s") * NC + lax.axis_index("c")
    iota = lax.iota(jnp.int32, L)
    col0 = iota * 2
    col1 = col0 + 1
    col2 = col0 + W
    col3 = col0 + (W + 1)
    blk_lo = wid * NBLK_W
    blk_hi = blk_lo + NBLK_W - 1

    def in_start(blk, buf, sem):
        img = blk // 4
        sub = blk % 4
        b_ = img // SC_C
        c_ = TC_C + (img % SC_C)
        off = ((b_ * C + c_) * H + sub * (2 * RB)) * W
        pltpu.async_copy(x_hbm.at[pl.ds(off, RB * 2 * W)], buf, sem)

    def in_wait(buf, sem):
        pltpu.make_async_copy(x_hbm.at[pl.ds(0, RB * 2 * W)], buf, sem).wait()

    def out_start(blk, buf, sem):
        img = blk // 4
        sub = blk % 4
        off = (img * OUT + sub * RB) * OUT
        pltpu.async_copy(buf, o_hbm.at[pl.ds(off, RB * OUT)], sem)

    def out_wait(buf, sem):
        pltpu.make_async_copy(buf, o_hbm.at[pl.ds(0, RB * OUT)], sem).wait()

    def compute(in_v, out_v):
        @plsc.parallel_loop(0, RB, unroll=2)
        def row_body(r):
            b0 = r * (2 * W)
            for g in range(GROUPS):
                win = in_v.at[pl.ds(b0 + g * (2 * L), W + 2 * L)]
                x0 = plsc.load_gather(win, [col0])
                x1 = plsc.load_gather(win, [col1])
                x2 = plsc.load_gather(win, [col2])
                x3 = plsc.load_gather(win, [col3])
                out_v[pl.ds(r * OUT + g * L, L)] = _fuzzy_math(x0, x1, x2, x3)

    in_start(blk_lo, in0, si0)
    in_start(blk_lo + 1, in1, si1)

    def pair_body(j, carry):
        b = blk_lo + j * 2
        in_wait(in0, si0)

        @pl.when(j > 0)
        def _():
            out_wait(out0, so0)

        compute(in0, out0)
        out_start(b, out0, so0)
        in_start(jnp.minimum(b + 2, blk_hi), in0, si0)

        in_wait(in1, si1)

        @pl.when(j > 0)
        def _():
            out_wait(out1, so1)

        compute(in1, out1)
        out_start(b + 1, out1, so1)
        in_start(jnp.minimum(b + 3, blk_hi), in1, si1)
        return carry

    lax.fori_loop(0, NBLK_W // 2, pair_body, 0)
    out_wait(out0, so0)
    out_wait(out1, so1)
    in_wait(in0, si0)
    in_wait(in1, si1)


_mesh = plsc.VectorSubcoreMesh(core_axis_name="c", subcore_axis_name="s",
                               num_cores=NC, num_subcores=NS)

_fuzzy_call = pl.kernel(
    _fuzzy_body,
    out_type=jax.ShapeDtypeStruct((SC_IMG * OUT * OUT,), jnp.float32),
    mesh=_mesh,
    scratch_types=[
        pltpu.VMEM((RB * 2 * W,), jnp.float32),
        pltpu.VMEM((RB * 2 * W,), jnp.float32),
        pltpu.VMEM((RB * OUT,), jnp.float32),
        pltpu.VMEM((RB * OUT,), jnp.float32),
        pltpu.SemaphoreType.DMA,
        pltpu.SemaphoreType.DMA,
        pltpu.SemaphoreType.DMA,
        pltpu.SemaphoreType.DMA,
    ],
    compiler_params=pltpu.CompilerParams(needs_layout_passes=False),
)


def _tc_body(x_ref, se_ref, so_ref, o_ref):
    se = se_ref[...]
    so = so_ref[...]
    x = x_ref[0].reshape(TC_C * 2 * HB, W)
    xh = x.astype(jnp.bfloat16).astype(jnp.float32)
    xr = x - xh
    xm = xr.astype(jnp.bfloat16).astype(jnp.float32)
    xl = xr - xm
    def dot3(sel):
        d = lambda a: jax.lax.dot(a, sel, preferred_element_type=jnp.float32)
        return (d(xh) + d(xm)) + d(xl)
    ye = dot3(se).reshape(TC_C, 2 * HB, OUT)
    yo = dot3(so).reshape(TC_C, 2 * HB, OUT)
    for hb in range(HB):
        x0 = ye[:, 2 * hb, :]
        x1 = yo[:, 2 * hb, :]
        x2 = ye[:, 2 * hb + 1, :]
        x3 = yo[:, 2 * hb + 1, :]
        o_ref[0, :, hb, :] = _fuzzy_math(x0, x1, x2, x3)


_tc_call = pl.pallas_call(
    _tc_body,
    grid=(B, OUT // HB),
    in_specs=[
        pl.BlockSpec((1, TC_C, 2 * HB, W), lambda b, h: (b, 0, h, 0)),
        pl.BlockSpec((W, OUT), lambda b, h: (0, 0)),
        pl.BlockSpec((W, OUT), lambda b, h: (0, 0)),
    ],
    out_specs=pl.BlockSpec((1, TC_C, HB, OUT), lambda b, h: (b, 0, h, 0)),
    out_shape=jax.ShapeDtypeStruct((B, TC_C, OUT, OUT), jnp.float32),
)

_SEL_EVEN = (np.arange(W)[:, None] == 2 * np.arange(OUT)[None, :]
             ).astype(np.float32)
_SEL_ODD = (np.arange(W)[:, None] == 2 * np.arange(OUT)[None, :] + 1
            ).astype(np.float32)


@jax.jit
def kernel(x):
    sc_out = _fuzzy_call(x.reshape(-1))
    tc_out = _tc_call(x, jnp.asarray(_SEL_EVEN), jnp.asarray(_SEL_ODD))
    return jnp.concatenate(
        [tc_out, sc_out.reshape(B, SC_C, OUT, OUT)], axis=1)


# speedup vs baseline: 1.3349x; 1.3349x over previous
import functools

import jax
import jax.numpy as jnp
import numpy as np
from jax import lax
from jax.experimental import pallas as pl
from jax.experimental.pallas import tpu as pltpu
from jax.experimental.pallas import tpu_sc as plsc

NC = 2
NS = 16
NW = NC * NS
L = 16

H = 224
W = 224
OUT = H // 2
B = 4
C = 96
TC_C = 32
SC_C = C - TC_C
SC_IMG = B * SC_C
RB = 16
SUBS = OUT // RB
NBLK_W = SC_IMG * SUBS // NW
GROUPS = OUT // L
HB = 8


def _fuzzy_math(x0, x1, x2, x3):
    m_inner = (x1 + x2) * 0.5
    m_all = ((x0 + x1) + (x2 + x3)) * 0.25
    v_avg = (m_inner + m_all) * 0.5
    w0 = jnp.abs(x0 - v_avg)
    w1 = jnp.abs(x1 - v_avg)
    w2 = jnp.abs(x2 - v_avg)
    w3 = jnp.abs(x3 - v_avg)
    s0 = (w1 + w2) * 0.5 + 1e-4
    s1 = ((w0 + w1) + (w2 + w3)) * 0.25 + 1e-4

    r0 = 1.0 / s0
    r1 = 1.0 / s1
    c0 = (r0 * r0) * -0.5
    c1 = (r1 * r1) * -0.5

    def gauss(xm, mu, c):
        d = xm - mu
        return jnp.exp((d * d) * c)

    p00 = gauss(x0, m_inner, c0)
    p0i = gauss(x1, m_inner, c0)
    p03 = gauss(x3, m_inner, c0)
    p10 = gauss(x0, m_all, c1)
    p11 = gauss(x1, m_all, c1)
    p12 = gauss(x2, m_all, c1)
    p13 = gauss(x3, m_all, c1)

    thresh = jnp.minimum(
        jnp.minimum(jnp.maximum(p00, p10), jnp.maximum(p0i, p11)),
        jnp.minimum(jnp.maximum(p0i, p12), jnp.maximum(p03, p13)))
    q0 = p00 + p10
    q1 = p0i + p11
    q2 = p0i + p12
    q3 = p03 + p13

    primary = q1 >= (thresh + thresh)
    secondary = jnp.logical_and(jnp.logical_not(primary), s1 < 0.001)
    num = (q0 * x0 + q1 * x1) + (q2 * x2 + q3 * x3)
    den = (q0 + q1) + (q2 + q3)
    denoised = num / den
    return jnp.where(primary, m_all, jnp.where(secondary, v_avg, denoised))


def _fuzzy_body(x_hbm, o_hbm, in0, in1, out0, out1, si0, si1, so0, so1):
    wid = lax.axis_index("s") * NC + lax.axis_index("c")
    iota = lax.iota(jnp.int32, L)
    col0 = iota * 2
    col1 = col0 + 1
    blk_lo = wid * NBLK_W
    blk_hi = blk_lo + NBLK_W - 1

    def in_start(blk, buf, sem):
        img = blk // SUBS
        sub = blk % SUBS
        b_ = img // SC_C
        c_ = TC_C + (img % SC_C)
        pltpu.async_copy(
            x_hbm.at[b_, c_, pl.ds(sub * (2 * RB), 2 * RB), :], buf, sem)

    def in_wait(buf, sem):
        pltpu.make_async_copy(
            x_hbm.at[0, 0, pl.ds(0, 2 * RB), :], buf, sem).wait()

    def out_start(blk, buf, sem):
        img = blk // SUBS
        sub = blk % SUBS
        b_ = img // SC_C
        c_ = img % SC_C
        pltpu.async_copy(
            buf, o_hbm.at[b_, c_, pl.ds(sub * RB, RB), :], sem)

    def out_wait(buf, sem):
        pltpu.make_async_copy(
            buf, o_hbm.at[0, 0, pl.ds(0, RB), :], sem).wait()

    def compute(in_v, out_v):
        @plsc.parallel_loop(0, RB, unroll=2)
        def row_body(r):
            for g in range(GROUPS):
                w0 = in_v.at[2 * r, pl.ds(g * (2 * L), 2 * L)]
                w1 = in_v.at[2 * r + 1, pl.ds(g * (2 * L), 2 * L)]
                x0 = plsc.load_gather(w0, [col0])
                x1 = plsc.load_gather(w0, [col1])
                x2 = plsc.load_gather(w1, [col0])
                x3 = plsc.load_gather(w1, [col1])
                out_v[r, pl.ds(g * L, L)] = _fuzzy_math(x0, x1, x2, x3)

    in_start(blk_lo, in0, si0)
    in_start(blk_lo + 1, in1, si1)

    def pair_body(j, carry):
        b = blk_lo + j * 2
        in_wait(in0, si0)

        @pl.when(j > 0)
        def _():
            out_wait(out0, so0)

        compute(in0, out0)
        out_start(b, out0, so0)
        in_start(jnp.minimum(b + 2, blk_hi), in0, si0)

        in_wait(in1, si1)

        @pl.when(j > 0)
        def _():
            out_wait(out1, so1)

        compute(in1, out1)
        out_start(b + 1, out1, so1)
        in_start(jnp.minimum(b + 3, blk_hi), in1, si1)
        return carry

    lax.fori_loop(0, NBLK_W // 2, pair_body, 0)
    out_wait(out0, so0)
    out_wait(out1, so1)
    in_wait(in0, si0)
    in_wait(in1, si1)


_mesh = plsc.VectorSubcoreMesh(core_axis_name="c", subcore_axis_name="s",
                               num_cores=NC, num_subcores=NS)

_fuzzy_call = pl.kernel(
    _fuzzy_body,
    out_type=jax.ShapeDtypeStruct((B, SC_C, OUT, OUT), jnp.float32),
    mesh=_mesh,
    scratch_types=[
        pltpu.VMEM((2 * RB, W), jnp.float32),
        pltpu.VMEM((2 * RB, W), jnp.float32),
        pltpu.VMEM((RB, OUT), jnp.float32),
        pltpu.VMEM((RB, OUT), jnp.float32),
        pltpu.SemaphoreType.DMA,
        pltpu.SemaphoreType.DMA,
        pltpu.SemaphoreType.DMA,
        pltpu.SemaphoreType.DMA,
    ],
    compiler_params=pltpu.CompilerParams(needs_layout_passes=False),
)


def _tc_body(x_ref, se_ref, so_ref, o_ref):
    se = se_ref[...]
    so = so_ref[...]
    x = x_ref[0].reshape(TC_C * 2 * HB, W)
    xh = x.astype(jnp.bfloat16).astype(jnp.float32)
    xr = x - xh
    xm = xr.astype(jnp.bfloat16).astype(jnp.float32)
    xl = xr - xm
    def dot3(sel):
        d = lambda a: jax.lax.dot(a, sel, preferred_element_type=jnp.float32)
        return (d(xh) + d(xm)) + d(xl)
    ye = dot3(se).reshape(TC_C, 2 * HB, OUT)
    yo = dot3(so).reshape(TC_C, 2 * HB, OUT)
    for hb in range(HB):
        x0 = ye[:, 2 * hb, :]
        x1 = yo[:, 2 * hb, :]
        x2 = ye[:, 2 * hb + 1, :]
        x3 = yo[:, 2 * hb + 1, :]
        o_ref[0, :, hb, :] = _fuzzy_math(x0, x1, x2, x3)


_tc_call = pl.pallas_call(
    _tc_body,
    grid=(B, OUT // HB),
    in_specs=[
        pl.BlockSpec((1, TC_C, 2 * HB, W), lambda b, h: (b, 0, h, 0)),
        pl.BlockSpec((W, OUT), lambda b, h: (0, 0)),
        pl.BlockSpec((W, OUT), lambda b, h: (0, 0)),
    ],
    out_specs=pl.BlockSpec((1, TC_C, HB, OUT), lambda b, h: (b, 0, h, 0)),
    out_shape=jax.ShapeDtypeStruct((B, TC_C, OUT, OUT), jnp.float32),
)

_SEL_EVEN = (np.arange(W)[:, None] == 2 * np.arange(OUT)[None, :]
             ).astype(np.float32)
_SEL_ODD = (np.arange(W)[:, None] == 2 * np.arange(OUT)[None, :] + 1
            ).astype(np.float32)


@jax.jit
def kernel(x):
    sc_out = _fuzzy_call(x)
    tc_out = _tc_call(x, jnp.asarray(_SEL_EVEN), jnp.asarray(_SEL_ODD))
    return jnp.concatenate([tc_out, sc_out], axis=1)


# speedup vs baseline: 1.3707x; 1.0268x over previous
import functools

import jax
import jax.numpy as jnp
import numpy as np
from jax import lax
from jax.experimental import pallas as pl
from jax.experimental.pallas import tpu as pltpu
from jax.experimental.pallas import tpu_sc as plsc

NC = 2
NS = 16
NW = NC * NS
L = 16

H = 224
W = 224
OUT = H // 2
B = 4
C = 96
TC_C = 40
SC_C = C - TC_C
SC_IMG = B * SC_C
RB = 8
SUBS = OUT // RB
NBLK_W = SC_IMG * SUBS // NW
GROUPS = OUT // L
HB = 8


def _fuzzy_math(x0, x1, x2, x3):
    m_inner = (x1 + x2) * 0.5
    m_all = ((x0 + x1) + (x2 + x3)) * 0.25
    v_avg = (m_inner + m_all) * 0.5
    w0 = jnp.abs(x0 - v_avg)
    w1 = jnp.abs(x1 - v_avg)
    w2 = jnp.abs(x2 - v_avg)
    w3 = jnp.abs(x3 - v_avg)
    s0 = (w1 + w2) * 0.5 + 1e-4
    s1 = ((w0 + w1) + (w2 + w3)) * 0.25 + 1e-4

    r0 = 1.0 / s0
    r1 = 1.0 / s1
    c0 = (r0 * r0) * -0.5
    c1 = (r1 * r1) * -0.5

    def gauss(xm, mu, c):
        d = xm - mu
        return jnp.exp((d * d) * c)

    p00 = gauss(x0, m_inner, c0)
    p0i = gauss(x1, m_inner, c0)
    p03 = gauss(x3, m_inner, c0)
    p10 = gauss(x0, m_all, c1)
    p11 = gauss(x1, m_all, c1)
    p12 = gauss(x2, m_all, c1)
    p13 = gauss(x3, m_all, c1)

    thresh = jnp.minimum(
        jnp.minimum(jnp.maximum(p00, p10), jnp.maximum(p0i, p11)),
        jnp.minimum(jnp.maximum(p0i, p12), jnp.maximum(p03, p13)))
    q0 = p00 + p10
    q1 = p0i + p11
    q2 = p0i + p12
    q3 = p03 + p13

    primary = q1 >= (thresh + thresh)
    secondary = jnp.logical_and(jnp.logical_not(primary), s1 < 0.001)
    num = (q0 * x0 + q1 * x1) + (q2 * x2 + q3 * x3)
    den = (q0 + q1) + (q2 + q3)
    denoised = num / den
    return jnp.where(primary, m_all, jnp.where(secondary, v_avg, denoised))


def _fuzzy_body(x_hbm, o_hbm, in0, in1, out0, out1, si0, si1, so0, so1):
    wid = lax.axis_index("s") * NC + lax.axis_index("c")
    iota = lax.iota(jnp.int32, L)
    col0 = iota * 2
    col1 = col0 + 1
    blk_lo = wid * NBLK_W
    blk_hi = blk_lo + NBLK_W - 1

    def in_start(blk, buf, sem):
        img = blk // SUBS
        sub = blk % SUBS
        b_ = img // SC_C
        c_ = TC_C + (img % SC_C)
        pltpu.async_copy(
            x_hbm.at[b_, c_, pl.ds(sub * (2 * RB), 2 * RB), :], buf, sem)

    def in_wait(buf, sem):
        pltpu.make_async_copy(
            x_hbm.at[0, 0, pl.ds(0, 2 * RB), :], buf, sem).wait()

    def out_start(blk, buf, sem):
        img = blk // SUBS
        sub = blk % SUBS
        b_ = img // SC_C
        c_ = img % SC_C
        pltpu.async_copy(
            buf, o_hbm.at[b_, c_, pl.ds(sub * RB, RB), :], sem)

    def out_wait(buf, sem):
        pltpu.make_async_copy(
            buf, o_hbm.at[0, 0, pl.ds(0, RB), :], sem).wait()

    def compute(in_v, out_v):
        @plsc.parallel_loop(0, RB, unroll=2)
        def row_body(r):
            for g in range(GROUPS):
                w0 = in_v.at[2 * r, pl.ds(g * (2 * L), 2 * L)]
                w1 = in_v.at[2 * r + 1, pl.ds(g * (2 * L), 2 * L)]
                x0 = plsc.load_gather(w0, [col0])
                x1 = plsc.load_gather(w0, [col1])
                x2 = plsc.load_gather(w1, [col0])
                x3 = plsc.load_gather(w1, [col1])
                out_v[r, pl.ds(g * L, L)] = _fuzzy_math(x0, x1, x2, x3)

    in_start(blk_lo, in0, si0)
    in_start(blk_lo + 1, in1, si1)

    def pair_body(j, carry):
        b = blk_lo + j * 2
        in_wait(in0, si0)

        @pl.when(j > 0)
        def _():
            out_wait(out0, so0)

        compute(in0, out0)
        out_start(b, out0, so0)
        in_start(jnp.minimum(b + 2, blk_hi), in0, si0)

        in_wait(in1, si1)

        @pl.when(j > 0)
        def _():
            out_wait(out1, so1)

        compute(in1, out1)
        out_start(b + 1, out1, so1)
        in_start(jnp.minimum(b + 3, blk_hi), in1, si1)
        return carry

    lax.fori_loop(0, NBLK_W // 2, pair_body, 0)
    out_wait(out0, so0)
    out_wait(out1, so1)
    in_wait(in0, si0)
    in_wait(in1, si1)


_mesh = plsc.VectorSubcoreMesh(core_axis_name="c", subcore_axis_name="s",
                               num_cores=NC, num_subcores=NS)

_fuzzy_call = pl.kernel(
    _fuzzy_body,
    out_type=jax.ShapeDtypeStruct((B, SC_C, OUT, OUT), jnp.float32),
    mesh=_mesh,
    scratch_types=[
        pltpu.VMEM((2 * RB, W), jnp.float32),
        pltpu.VMEM((2 * RB, W), jnp.float32),
        pltpu.VMEM((RB, OUT), jnp.float32),
        pltpu.VMEM((RB, OUT), jnp.float32),
        pltpu.SemaphoreType.DMA,
        pltpu.SemaphoreType.DMA,
        pltpu.SemaphoreType.DMA,
        pltpu.SemaphoreType.DMA,
    ],
    compiler_params=pltpu.CompilerParams(needs_layout_passes=False),
)


def _tc_body(x_ref, se_ref, so_ref, o_ref):
    se = se_ref[...]
    so = so_ref[...]
    x = x_ref[0].reshape(TC_C * 2 * HB, W)
    xh = x.astype(jnp.bfloat16).astype(jnp.float32)
    xr = x - xh
    xm = xr.astype(jnp.bfloat16).astype(jnp.float32)
    xl = xr - xm
    def dot3(sel):
        d = lambda a: jax.lax.dot(a, sel, preferred_element_type=jnp.float32)
        return (d(xh) + d(xm)) + d(xl)
    ye = dot3(se).reshape(TC_C, 2 * HB, OUT)
    yo = dot3(so).reshape(TC_C, 2 * HB, OUT)
    for hb in range(HB):
        x0 = ye[:, 2 * hb, :]
        x1 = yo[:, 2 * hb, :]
        x2 = ye[:, 2 * hb + 1, :]
        x3 = yo[:, 2 * hb + 1, :]
        o_ref[0, :, hb, :] = _fuzzy_math(x0, x1, x2, x3)


_tc_call = pl.pallas_call(
    _tc_body,
    grid=(B, OUT // HB),
    in_specs=[
        pl.BlockSpec((1, TC_C, 2 * HB, W), lambda b, h: (b, 0, h, 0)),
        pl.BlockSpec((W, OUT), lambda b, h: (0, 0)),
        pl.BlockSpec((W, OUT), lambda b, h: (0, 0)),
    ],
    out_specs=pl.BlockSpec((1, TC_C, HB, OUT), lambda b, h: (b, 0, h, 0)),
    out_shape=jax.ShapeDtypeStruct((B, TC_C, OUT, OUT), jnp.float32),
)

_SEL_EVEN = (np.arange(W)[:, None] == 2 * np.arange(OUT)[None, :]
             ).astype(np.float32)
_SEL_ODD = (np.arange(W)[:, None] == 2 * np.arange(OUT)[None, :] + 1
            ).astype(np.float32)


@jax.jit
def kernel(x):
    sc_out = _fuzzy_call(x)
    tc_out = _tc_call(x, jnp.asarray(_SEL_EVEN), jnp.asarray(_SEL_ODD))
    return jnp.concatenate([tc_out, sc_out], axis=1)
